# Initial kernel scaffold; baseline (speedup 1.0000x reference)
import jax
import jax.numpy as jnp
from jax.experimental import pallas as pl


def kernel(x, edge_index, W1l, b1l, W1r, W2l, b2l, W2r):
    raise NotImplementedError("write your pallas kernel here")


# speedup vs baseline: 4.6907x; 4.6907x over previous
import jax
import jax.numpy as jnp
from jax import lax
from jax.experimental import pallas as pl
from jax.experimental.pallas import tpu as pltpu
from jax.experimental.pallas import tpu_sc as plsc

_N = 10000
_E = 320000
_F = 128
_NC = 2
_NS = 16
_NPAD = 10240
_BATCH = 80
_EPC = _E // _NC
_EPT = _EPC // _NS
_STEPS = _EPT // _BATCH
_RPT = _NPAD // _NS


def _zero_buf(buf, nrows, val=0.0):
    def zb(i, carry):
        buf[i // 8, pl.ds((i % 8) * 16, 16)] = jnp.full((16,), val, jnp.float32)
        return carry
    lax.fori_loop(0, nrows * 8, zb, 0)


def _mesh():
    return plsc.VectorSubcoreMesh(core_axis_name="c", subcore_axis_name="s",
                                  num_cores=_NC, num_subcores=_NS)


def _make_agg():
    outs = (jax.ShapeDtypeStruct((_NPAD, _F), jnp.float32),
            jax.ShapeDtypeStruct((_NPAD, _F), jnp.float32))
    scratch = (pltpu.VMEM((_BATCH,), jnp.int32),
               pltpu.VMEM((_BATCH,), jnp.int32),
               pltpu.VMEM((_BATCH, _F), jnp.float32),
               pltpu.VMEM_SHARED((_NPAD, _F), jnp.float32),
               pltpu.SemaphoreType.DMA)

    def body(x, src, dst, sum0, sum1, idx_s, idx_d, rows, acc_sh, sem):
        c = lax.axis_index("c")
        s = lax.axis_index("s")

        _zero_buf(rows, _BATCH)
        for k in range(_RPT // _BATCH):
            base_r = s * _RPT + k * _BATCH
            pltpu.sync_copy(rows, acc_sh.at[pl.ds(base_r, _BATCH)])

        plsc.subcore_barrier()

        def step(t, carry):
            base = c * _EPC + s * _EPT + t * _BATCH
            pltpu.sync_copy(src.at[pl.ds(base, _BATCH)], idx_s)
            pltpu.sync_copy(dst.at[pl.ds(base, _BATCH)], idx_d)
            pltpu.async_copy(x.at[idx_s], rows, sem).wait()
            pltpu.sync_copy(rows, acc_sh.at[idx_d], add=True)
            return carry
        lax.fori_loop(0, _STEPS, step, 0)

        plsc.subcore_barrier()

        for k in range(_RPT // _BATCH):
            base_r = s * _RPT + k * _BATCH
            pltpu.sync_copy(acc_sh.at[pl.ds(base_r, _BATCH)], rows)

            @pl.when(c == 0)
            def _w0():
                pltpu.sync_copy(rows, sum0.at[pl.ds(base_r, _BATCH)])

            @pl.when(c != 0)
            def _w1():
                pltpu.sync_copy(rows, sum1.at[pl.ds(base_r, _BATCH)])

    return pl.kernel(body, out_type=outs, mesh=_mesh(),
                     scratch_types=scratch)


def _make_deg():
    outs = (jax.ShapeDtypeStruct((_NPAD, _F), jnp.float32),
            jax.ShapeDtypeStruct((_NPAD, _F), jnp.float32))
    scratch = (pltpu.VMEM((_BATCH,), jnp.int32),
               pltpu.VMEM((_BATCH, _F), jnp.float32),
               pltpu.VMEM_SHARED((_NPAD, _F), jnp.float32))

    def body(dst, deg0, deg1, idx_d, ones_r, acc_sh):
        c = lax.axis_index("c")
        s = lax.axis_index("s")

        _zero_buf(ones_r, _BATCH)
        for k in range(_RPT // _BATCH):
            base_r = s * _RPT + k * _BATCH
            pltpu.sync_copy(ones_r, acc_sh.at[pl.ds(base_r, _BATCH)])
        _zero_buf(ones_r, _BATCH, val=1.0)

        plsc.subcore_barrier()

        def step(t, carry):
            base = c * _EPC + s * _EPT + t * _BATCH
            pltpu.sync_copy(dst.at[pl.ds(base, _BATCH)], idx_d)
            pltpu.sync_copy(ones_r, acc_sh.at[idx_d], add=True)
            return carry
        lax.fori_loop(0, _STEPS, step, 0)

        plsc.subcore_barrier()

        for k in range(_RPT // _BATCH):
            base_r = s * _RPT + k * _BATCH
            pltpu.sync_copy(acc_sh.at[pl.ds(base_r, _BATCH)], ones_r)

            @pl.when(c == 0)
            def _w0():
                pltpu.sync_copy(ones_r, deg0.at[pl.ds(base_r, _BATCH)])

            @pl.when(c != 0)
            def _w1():
                pltpu.sync_copy(ones_r, deg1.at[pl.ds(base_r, _BATCH)])

    return pl.kernel(body, out_type=outs, mesh=_mesh(),
                     scratch_types=scratch)


def _make_combine(relu):
    rows = 1000

    def body(sum0_ref, sum1_ref, deg0_ref, deg1_ref, x_ref, wl_ref, bl_ref,
             wr_ref, out_ref):
        recip = 1.0 / jnp.maximum(deg0_ref[...] + deg1_ref[...], 1.0)
        mean = (sum0_ref[...] + sum1_ref[...]) * recip
        acc = lax.dot_general(mean, wl_ref[...], (((1,), (1,)), ((), ())),
                              preferred_element_type=jnp.float32)
        acc = acc + lax.dot_general(x_ref[...], wr_ref[...],
                                    (((1,), (1,)), ((), ())),
                                    preferred_element_type=jnp.float32)
        acc = acc + bl_ref[...]
        out_ref[...] = jnp.maximum(acc, 0.0) if relu else acc

    return pl.pallas_call(
        body,
        grid=(_N // rows,),
        in_specs=[
            pl.BlockSpec((rows, _F), lambda i: (i, 0)),
            pl.BlockSpec((rows, _F), lambda i: (i, 0)),
            pl.BlockSpec((rows, _F), lambda i: (i, 0)),
            pl.BlockSpec((rows, _F), lambda i: (i, 0)),
            pl.BlockSpec((rows, _F), lambda i: (i, 0)),
            pl.BlockSpec((_F, _F), lambda i: (0, 0)),
            pl.BlockSpec((1, _F), lambda i: (0, 0)),
            pl.BlockSpec((_F, _F), lambda i: (0, 0)),
        ],
        out_specs=pl.BlockSpec((rows, _F), lambda i: (i, 0)),
        out_shape=jax.ShapeDtypeStruct((_N, _F), jnp.float32),
    )


_agg = _make_agg()
_deg = _make_deg()
_combine_relu = _make_combine(True)
_combine = _make_combine(False)


def kernel(x, edge_index, W1l, b1l, W1r, W2l, b2l, W2r):
    src = edge_index[0]
    dst = edge_index[1]
    deg0, deg1 = _deg(dst)
    sum0, sum1 = _agg(x, src, dst)
    h = _combine_relu(sum0[:_N], sum1[:_N], deg0[:_N], deg1[:_N], x,
                      W1l, b1l.reshape(1, _F), W1r)
    t0, t1 = _agg(h, src, dst)
    out = _combine(t0[:_N], t1[:_N], deg0[:_N], deg1[:_N], h,
                   W2l, b2l.reshape(1, _F), W2r)
    return out


# speedup vs baseline: 9.0277x; 1.9246x over previous
import jax
import jax.numpy as jnp
from jax import lax
from jax.experimental import pallas as pl
from jax.experimental.pallas import tpu as pltpu
from jax.experimental.pallas import tpu_sc as plsc

_N = 10000
_E = 320000
_F = 128
_NC = 2
_NS = 16
_NW = _NC * _NS
_NPAD = 10240
_EPT = _E // _NW
_B = 40
_STEPS = _EPT // _B
_K = 5
_GROUPS = _STEPS // _K
_RPT = _NPAD // _NS


def _zero_buf(buf, nrows, val=0.0):
    def zb(i, carry):
        buf[i // 8, pl.ds((i % 8) * 16, 16)] = jnp.full((16,), val, jnp.float32)
        return carry
    lax.fori_loop(0, nrows * 8, zb, 0)


def _mesh():
    return plsc.VectorSubcoreMesh(core_axis_name="c", subcore_axis_name="s",
                                  num_cores=_NC, num_subcores=_NS)


def _make_agg():
    outs = (jax.ShapeDtypeStruct((_NPAD, _F), jnp.float32),
            jax.ShapeDtypeStruct((_NPAD, _F), jnp.float32))
    scratch = (
        (pltpu.VMEM((_EPT,), jnp.int32),)
        + (pltpu.VMEM((_B,), jnp.int32),) * _K
        + (pltpu.VMEM((_B, _F), jnp.float32),) * _K
        + (pltpu.VMEM_SHARED((_NPAD, _F), jnp.float32),)
        + (pltpu.SemaphoreType.DMA,) * (3 * _K)
    )

    def body(x, src, dst, sum0, sum1, isrc, *rest):
        idxd = rest[0:_K]
        rows = rest[_K:2 * _K]
        acc_sh = rest[2 * _K]
        sem_id = rest[2 * _K + 1:2 * _K + 1 + _K]
        sem_g = rest[2 * _K + 1 + _K:2 * _K + 1 + 2 * _K]
        sem_s = rest[2 * _K + 1 + 2 * _K:]
        c = lax.axis_index("c")
        s = lax.axis_index("s")
        w = c * _NS + s
        ebase = w * _EPT

        pltpu.sync_copy(src.at[pl.ds(ebase, _EPT)], isrc)
        _zero_buf(rows[0], _B)
        for k in range(_RPT // _B):
            pltpu.sync_copy(rows[0], acc_sh.at[pl.ds(s * _RPT + k * _B, _B)])

        plsc.subcore_barrier()

        for b in range(3):
            pltpu.async_copy(dst.at[pl.ds(ebase + b * _B, _B)], idxd[b],
                             sem_id[b])
        for b in range(2):
            pltpu.async_copy(x.at[isrc.at[pl.ds(b * _B, _B)]], rows[b],
                             sem_g[b])

        def group(g, carry):
            for b in range(_K):
                t = g * _K + b
                pltpu.make_async_copy(dst.at[pl.ds(ebase, _B)], idxd[b],
                                      sem_id[b]).wait()
                pltpu.make_async_copy(x.at[isrc.at[pl.ds(0, _B)]], rows[b],
                                      sem_g[b]).wait()
                pltpu.async_copy(rows[b], acc_sh.at[idxd[b]], sem_s[b],
                                 add=True)

                bp = (b - 2) % _K

                def free_slot(bp=bp):
                    pltpu.make_async_copy(rows[bp], acc_sh.at[idxd[bp]],
                                          sem_s[bp]).wait()

                if b >= 2:
                    free_slot()
                else:
                    @pl.when(g > 0)
                    def _fs():
                        free_slot()

                def pre_idx(bp=bp, tg=t + 3):
                    pltpu.async_copy(
                        dst.at[pl.ds(ebase + tg * _B, _B)], idxd[bp],
                        sem_id[bp])

                if b < 2:
                    pre_idx()
                else:
                    @pl.when(g < _GROUPS - 1)
                    def _pi():
                        pre_idx()

                bg = (b + 2) % _K

                def pre_g(bg=bg, tg=t + 2):
                    pltpu.async_copy(x.at[isrc.at[pl.ds(tg * _B, _B)]],
                                     rows[bg], sem_g[bg])

                if b < 3:
                    pre_g()
                else:
                    @pl.when(g < _GROUPS - 1)
                    def _pg():
                        pre_g()
            return carry
        lax.fori_loop(0, _GROUPS, group, 0)

        for bp in (3, 4):
            pltpu.make_async_copy(rows[bp], acc_sh.at[idxd[bp]],
                                  sem_s[bp]).wait()

        plsc.subcore_barrier()

        for k in range(_RPT // _B):
            base_r = s * _RPT + k * _B
            pltpu.sync_copy(acc_sh.at[pl.ds(base_r, _B)], rows[0])

            @pl.when(c == 0)
            def _w0():
                pltpu.sync_copy(rows[0], sum0.at[pl.ds(base_r, _B)])

            @pl.when(c != 0)
            def _w1():
                pltpu.sync_copy(rows[0], sum1.at[pl.ds(base_r, _B)])

    return pl.kernel(body, out_type=outs, mesh=_mesh(),
                     scratch_types=scratch)


def _make_deg():
    outs = (jax.ShapeDtypeStruct((_NPAD, _F), jnp.float32),
            jax.ShapeDtypeStruct((_NPAD, _F), jnp.float32))
    scratch = (
        (pltpu.VMEM((_B,), jnp.int32),) * _K
        + (pltpu.VMEM((_B, _F), jnp.float32),)
        + (pltpu.VMEM_SHARED((_NPAD, _F), jnp.float32),)
        + (pltpu.SemaphoreType.DMA,) * (2 * _K)
    )

    def body(dst, deg0, deg1, *rest):
        idxd = rest[0:_K]
        ones_r = rest[_K]
        acc_sh = rest[_K + 1]
        sem_id = rest[_K + 2:_K + 2 + _K]
        sem_s = rest[_K + 2 + _K:]
        c = lax.axis_index("c")
        s = lax.axis_index("s")
        w = c * _NS + s
        ebase = w * _EPT

        _zero_buf(ones_r, _B)
        for k in range(_RPT // _B):
            pltpu.sync_copy(ones_r, acc_sh.at[pl.ds(s * _RPT + k * _B, _B)])
        _zero_buf(ones_r, _B, val=1.0)

        plsc.subcore_barrier()

        for b in range(3):
            pltpu.async_copy(dst.at[pl.ds(ebase + b * _B, _B)], idxd[b],
                             sem_id[b])

        def group(g, carry):
            for b in range(_K):
                t = g * _K + b
                pltpu.make_async_copy(dst.at[pl.ds(ebase, _B)], idxd[b],
                                      sem_id[b]).wait()
                pltpu.async_copy(ones_r, acc_sh.at[idxd[b]], sem_s[b],
                                 add=True)

                bp = (b - 2) % _K

                def free_slot(bp=bp):
                    pltpu.make_async_copy(ones_r, acc_sh.at[idxd[bp]],
                                          sem_s[bp]).wait()

                if b >= 2:
                    free_slot()
                else:
                    @pl.when(g > 0)
                    def _fs():
                        free_slot()

                def pre_idx(bp=bp, tg=t + 3):
                    pltpu.async_copy(
                        dst.at[pl.ds(ebase + tg * _B, _B)], idxd[bp],
                        sem_id[bp])

                if b < 2:
                    pre_idx()
                else:
                    @pl.when(g < _GROUPS - 1)
                    def _pi():
                        pre_idx()
            return carry
        lax.fori_loop(0, _GROUPS, group, 0)

        for bp in (3, 4):
            pltpu.make_async_copy(ones_r, acc_sh.at[idxd[bp]],
                                  sem_s[bp]).wait()

        plsc.subcore_barrier()

        for k in range(_RPT // _B):
            base_r = s * _RPT + k * _B
            pltpu.sync_copy(acc_sh.at[pl.ds(base_r, _B)], ones_r)

            @pl.when(c == 0)
            def _w0():
                pltpu.sync_copy(ones_r, deg0.at[pl.ds(base_r, _B)])

            @pl.when(c != 0)
            def _w1():
                pltpu.sync_copy(ones_r, deg1.at[pl.ds(base_r, _B)])

    return pl.kernel(body, out_type=outs, mesh=_mesh(),
                     scratch_types=scratch)


def _make_combine(relu):
    rows = 1000

    def body(sum0_ref, sum1_ref, deg0_ref, deg1_ref, x_ref, wl_ref, bl_ref,
             wr_ref, out_ref):
        recip = 1.0 / jnp.maximum(deg0_ref[...] + deg1_ref[...], 1.0)
        mean = (sum0_ref[...] + sum1_ref[...]) * recip
        acc = lax.dot_general(mean, wl_ref[...], (((1,), (1,)), ((), ())),
                              preferred_element_type=jnp.float32)
        acc = acc + lax.dot_general(x_ref[...], wr_ref[...],
                                    (((1,), (1,)), ((), ())),
                                    preferred_element_type=jnp.float32)
        acc = acc + bl_ref[...]
        out_ref[...] = jnp.maximum(acc, 0.0) if relu else acc

    return pl.pallas_call(
        body,
        grid=(_N // rows,),
        in_specs=[
            pl.BlockSpec((rows, _F), lambda i: (i, 0)),
            pl.BlockSpec((rows, _F), lambda i: (i, 0)),
            pl.BlockSpec((rows, _F), lambda i: (i, 0)),
            pl.BlockSpec((rows, _F), lambda i: (i, 0)),
            pl.BlockSpec((rows, _F), lambda i: (i, 0)),
            pl.BlockSpec((_F, _F), lambda i: (0, 0)),
            pl.BlockSpec((1, _F), lambda i: (0, 0)),
            pl.BlockSpec((_F, _F), lambda i: (0, 0)),
        ],
        out_specs=pl.BlockSpec((rows, _F), lambda i: (i, 0)),
        out_shape=jax.ShapeDtypeStruct((_N, _F), jnp.float32),
    )


_agg = _make_agg()
_deg = _make_deg()
_combine_relu = _make_combine(True)
_combine = _make_combine(False)


def kernel(x, edge_index, W1l, b1l, W1r, W2l, b2l, W2r):
    src = edge_index[0]
    dst = edge_index[1]
    deg0, deg1 = _deg(dst)
    sum0, sum1 = _agg(x, src, dst)
    h = _combine_relu(sum0[:_N], sum1[:_N], deg0[:_N], deg1[:_N], x,
                      W1l, b1l.reshape(1, _F), W1r)
    t0, t1 = _agg(h, src, dst)
    out = _combine(t0[:_N], t1[:_N], deg0[:_N], deg1[:_N], h,
                   W2l, b2l.reshape(1, _F), W2r)
    return out


# speedup vs baseline: 9.4032x; 1.0416x over previous
import jax
import jax.numpy as jnp
from jax import lax
from jax.experimental import pallas as pl
from jax.experimental.pallas import tpu as pltpu
from jax.experimental.pallas import tpu_sc as plsc

_N = 10000
_E = 320000
_F = 128
_NC = 2
_NS = 16
_NW = _NC * _NS
_NPAD = 10240
_EPT = _E // _NW
_B = 40
_STEPS = _EPT // _B
_K = 5
_GROUPS = _STEPS // _K
_RPT = _NPAD // _NS


def _zero_buf(buf, nrows, val=0.0):
    def zb(i, carry):
        buf[i // 8, pl.ds((i % 8) * 16, 16)] = jnp.full((16,), val, jnp.float32)
        return carry
    lax.fori_loop(0, nrows * 8, zb, 0)


def _mesh():
    return plsc.VectorSubcoreMesh(core_axis_name="c", subcore_axis_name="s",
                                  num_cores=_NC, num_subcores=_NS)


def _make_agg():
    outs = (jax.ShapeDtypeStruct((_NPAD, _F), jnp.float32),
            jax.ShapeDtypeStruct((_NPAD, _F), jnp.float32))
    scratch = (
        (pltpu.VMEM((_EPT,), jnp.int32),)
        + (pltpu.VMEM((_B,), jnp.int32),) * _K
        + (pltpu.VMEM((_B, _F), jnp.float32),) * _K
        + (pltpu.VMEM_SHARED((_NPAD, _F), jnp.float32),)
        + (pltpu.SemaphoreType.DMA,) * (3 * _K)
    )

    def body(x, src, dst, sum0, sum1, isrc, *rest):
        idxd = rest[0:_K]
        rows = rest[_K:2 * _K]
        acc_sh = rest[2 * _K]
        sem_id = rest[2 * _K + 1:2 * _K + 1 + _K]
        sem_g = rest[2 * _K + 1 + _K:2 * _K + 1 + 2 * _K]
        sem_s = rest[2 * _K + 1 + 2 * _K:]
        c = lax.axis_index("c")
        s = lax.axis_index("s")
        w = c * _NS + s
        ebase = w * _EPT

        pltpu.sync_copy(src.at[pl.ds(ebase, _EPT)], isrc)
        _zero_buf(rows[0], _B)
        for k in range(_RPT // _B):
            pltpu.sync_copy(rows[0], acc_sh.at[pl.ds(s * _RPT + k * _B, _B)])

        plsc.subcore_barrier()

        for b in range(2):
            pltpu.async_copy(dst.at[pl.ds(ebase + b * _B, _B)], idxd[b],
                             sem_id[b])
            pltpu.async_copy(x.at[isrc.at[pl.ds(b * _B, _B)]], rows[b],
                             sem_g[b])

        def group(g, carry):
            for b in range(_K):
                t = g * _K + b
                pltpu.make_async_copy(dst.at[pl.ds(ebase, _B)], idxd[b],
                                      sem_id[b]).wait()
                pltpu.make_async_copy(x.at[isrc.at[pl.ds(0, _B)]], rows[b],
                                      sem_g[b]).wait()
                pltpu.async_copy(rows[b], acc_sh.at[idxd[b]], sem_s[b],
                                 add=True)

                bp = (b + 2) % _K

                def free_slot(bp=bp):
                    pltpu.make_async_copy(rows[bp], acc_sh.at[idxd[bp]],
                                          sem_s[bp]).wait()

                def pre(bp=bp, tg=t + 2):
                    pltpu.async_copy(
                        dst.at[pl.ds(ebase + tg * _B, _B)], idxd[bp],
                        sem_id[bp])
                    pltpu.async_copy(x.at[isrc.at[pl.ds(tg * _B, _B)]],
                                     rows[bp], sem_g[bp])

                if b >= 3:
                    free_slot()

                    @pl.when(g < _GROUPS - 1)
                    def _pf():
                        pre()
                else:
                    @pl.when(g > 0)
                    def _fs():
                        free_slot()
                    pre()
            return carry
        lax.fori_loop(0, _GROUPS, group, 0)

        for bp in (2, 3, 4):
            pltpu.make_async_copy(rows[bp], acc_sh.at[idxd[bp]],
                                  sem_s[bp]).wait()

        plsc.subcore_barrier()

        for k in range(_RPT // _B):
            base_r = s * _RPT + k * _B
            pltpu.sync_copy(acc_sh.at[pl.ds(base_r, _B)], rows[0])

            @pl.when(c == 0)
            def _w0():
                pltpu.sync_copy(rows[0], sum0.at[pl.ds(base_r, _B)])

            @pl.when(c != 0)
            def _w1():
                pltpu.sync_copy(rows[0], sum1.at[pl.ds(base_r, _B)])

    return pl.kernel(body, out_type=outs, mesh=_mesh(),
                     scratch_types=scratch)


def _make_deg():
    outs = (jax.ShapeDtypeStruct((_NPAD, _F), jnp.float32),
            jax.ShapeDtypeStruct((_NPAD, _F), jnp.float32))
    scratch = (
        (pltpu.VMEM((_B,), jnp.int32),) * _K
        + (pltpu.VMEM((_B, _F), jnp.float32),)
        + (pltpu.VMEM_SHARED((_NPAD, _F), jnp.float32),)
        + (pltpu.SemaphoreType.DMA,) * (2 * _K)
    )

    def body(dst, deg0, deg1, *rest):
        idxd = rest[0:_K]
        ones_r = rest[_K]
        acc_sh = rest[_K + 1]
        sem_id = rest[_K + 2:_K + 2 + _K]
        sem_s = rest[_K + 2 + _K:]
        c = lax.axis_index("c")
        s = lax.axis_index("s")
        w = c * _NS + s
        ebase = w * _EPT

        _zero_buf(ones_r, _B)
        for k in range(_RPT // _B):
            pltpu.sync_copy(ones_r, acc_sh.at[pl.ds(s * _RPT + k * _B, _B)])
        _zero_buf(ones_r, _B, val=1.0)

        plsc.subcore_barrier()

        for b in range(2):
            pltpu.async_copy(dst.at[pl.ds(ebase + b * _B, _B)], idxd[b],
                             sem_id[b])

        def group(g, carry):
            for b in range(_K):
                t = g * _K + b
                pltpu.make_async_copy(dst.at[pl.ds(ebase, _B)], idxd[b],
                                      sem_id[b]).wait()
                pltpu.async_copy(ones_r, acc_sh.at[idxd[b]], sem_s[b],
                                 add=True)

                bp = (b + 2) % _K

                def free_slot(bp=bp):
                    pltpu.make_async_copy(ones_r, acc_sh.at[idxd[bp]],
                                          sem_s[bp]).wait()

                def pre_idx(bp=bp, tg=t + 2):
                    pltpu.async_copy(
                        dst.at[pl.ds(ebase + tg * _B, _B)], idxd[bp],
                        sem_id[bp])

                if b >= 3:
                    free_slot()

                    @pl.when(g < _GROUPS - 1)
                    def _pi():
                        pre_idx()
                else:
                    @pl.when(g > 0)
                    def _fs():
                        free_slot()
                    pre_idx()
            return carry
        lax.fori_loop(0, _GROUPS, group, 0)

        for bp in (2, 3, 4):
            pltpu.make_async_copy(ones_r, acc_sh.at[idxd[bp]],
                                  sem_s[bp]).wait()

        plsc.subcore_barrier()

        for k in range(_RPT // _B):
            base_r = s * _RPT + k * _B
            pltpu.sync_copy(acc_sh.at[pl.ds(base_r, _B)], ones_r)

            @pl.when(c == 0)
            def _w0():
                pltpu.sync_copy(ones_r, deg0.at[pl.ds(base_r, _B)])

            @pl.when(c != 0)
            def _w1():
                pltpu.sync_copy(ones_r, deg1.at[pl.ds(base_r, _B)])

    return pl.kernel(body, out_type=outs, mesh=_mesh(),
                     scratch_types=scratch)


def _make_combine(relu):
    rows = 1000

    def body(sum0_ref, sum1_ref, deg0_ref, deg1_ref, x_ref, wl_ref, bl_ref,
             wr_ref, out_ref):
        recip = 1.0 / jnp.maximum(deg0_ref[...] + deg1_ref[...], 1.0)
        mean = (sum0_ref[...] + sum1_ref[...]) * recip
        acc = lax.dot_general(mean, wl_ref[...], (((1,), (1,)), ((), ())),
                              preferred_element_type=jnp.float32)
        acc = acc + lax.dot_general(x_ref[...], wr_ref[...],
                                    (((1,), (1,)), ((), ())),
                                    preferred_element_type=jnp.float32)
        acc = acc + bl_ref[...]
        out_ref[...] = jnp.maximum(acc, 0.0) if relu else acc

    return pl.pallas_call(
        body,
        grid=(_N // rows,),
        in_specs=[
            pl.BlockSpec((rows, _F), lambda i: (i, 0)),
            pl.BlockSpec((rows, _F), lambda i: (i, 0)),
            pl.BlockSpec((rows, _F), lambda i: (i, 0)),
            pl.BlockSpec((rows, _F), lambda i: (i, 0)),
            pl.BlockSpec((rows, _F), lambda i: (i, 0)),
            pl.BlockSpec((_F, _F), lambda i: (0, 0)),
            pl.BlockSpec((1, _F), lambda i: (0, 0)),
            pl.BlockSpec((_F, _F), lambda i: (0, 0)),
        ],
        out_specs=pl.BlockSpec((rows, _F), lambda i: (i, 0)),
        out_shape=jax.ShapeDtypeStruct((_N, _F), jnp.float32),
    )


_agg = _make_agg()
_deg = _make_deg()
_combine_relu = _make_combine(True)
_combine = _make_combine(False)


def kernel(x, edge_index, W1l, b1l, W1r, W2l, b2l, W2r):
    src = edge_index[0]
    dst = edge_index[1]
    deg0, deg1 = _deg(dst)
    sum0, sum1 = _agg(x, src, dst)
    h = _combine_relu(sum0, sum1, deg0, deg1, x,
                      W1l, b1l.reshape(1, _F), W1r)
    t0, t1 = _agg(h, src, dst)
    out = _combine(t0, t1, deg0, deg1, h,
                   W2l, b2l.reshape(1, _F), W2r)
    return out


# speedup vs baseline: 10.6689x; 1.1346x over previous
import jax
import jax.numpy as jnp
from jax import lax
from jax.experimental import pallas as pl
from jax.experimental.pallas import tpu as pltpu
from jax.experimental.pallas import tpu_sc as plsc

_N = 10000
_E = 320000
_F = 128
_NC = 2
_NS = 16
_NW = _NC * _NS
_NPAD = 10240
_EPT = _E // _NW
_B = 80
_STEPS = _EPT // _B
_KR = 4
_KI = 6
_SLOT = 12
_GROUPS = 10
_RPT = _NPAD // _NS


def _zero_buf(buf, nrows, val=0.0):
    def zb(i, carry):
        buf[i // 8, pl.ds((i % 8) * 16, 16)] = jnp.full((16,), val, jnp.float32)
        return carry
    lax.fori_loop(0, nrows * 8, zb, 0)


def _mesh():
    return plsc.VectorSubcoreMesh(core_axis_name="c", subcore_axis_name="s",
                                  num_cores=_NC, num_subcores=_NS)


def _make_agg():
    outs = (jax.ShapeDtypeStruct((_NPAD, _F), jnp.float32),
            jax.ShapeDtypeStruct((_NPAD, _F), jnp.float32))
    scratch = (
        (pltpu.VMEM((_B,), jnp.int32),) * _KI
        + (pltpu.VMEM((_B,), jnp.int32),) * _KI
        + (pltpu.VMEM((_B, _F), jnp.float32),) * _KR
        + (pltpu.VMEM_SHARED((_NPAD, _F), jnp.float32),)
        + (pltpu.SemaphoreType.DMA,) * (2 * _KI + 2 * _KR)
    )

    def body(x, src, dst, sum0, sum1, *rest):
        isr = rest[0:_KI]
        idd = rest[_KI:2 * _KI]
        rows = rest[2 * _KI:2 * _KI + _KR]
        acc_sh = rest[2 * _KI + _KR]
        o = 2 * _KI + _KR + 1
        sem_is = rest[o:o + _KI]
        sem_id = rest[o + _KI:o + 2 * _KI]
        sem_g = rest[o + 2 * _KI:o + 2 * _KI + _KR]
        sem_s = rest[o + 2 * _KI + _KR:]
        c = lax.axis_index("c")
        s = lax.axis_index("s")
        w = c * _NS + s
        ebase = w * _EPT

        def idx_issue(t, bi):
            pltpu.async_copy(src.at[pl.ds(ebase + t * _B, _B)], isr[bi],
                             sem_is[bi])
            pltpu.async_copy(dst.at[pl.ds(ebase + t * _B, _B)], idd[bi],
                             sem_id[bi])

        def idx_s_wait(bi):
            pltpu.make_async_copy(src.at[pl.ds(ebase, _B)], isr[bi],
                                  sem_is[bi]).wait()

        def idx_d_wait(bi):
            pltpu.make_async_copy(dst.at[pl.ds(ebase, _B)], idd[bi],
                                  sem_id[bi]).wait()

        def gather_issue(bi, b):
            pltpu.async_copy(x.at[isr[bi]], rows[b], sem_g[b])

        def gather_wait(bi, b):
            pltpu.make_async_copy(x.at[isr[bi]], rows[b], sem_g[b]).wait()

        def scat_issue(bi, b):
            pltpu.async_copy(rows[b], acc_sh.at[idd[bi]], sem_s[b], add=True)

        def scat_wait(bi, b):
            pltpu.make_async_copy(rows[b], acc_sh.at[idd[bi]],
                                  sem_s[b]).wait()

        _zero_buf(rows[0], _B)
        for k in range(_RPT // _B):
            pltpu.sync_copy(rows[0], acc_sh.at[pl.ds(s * _RPT + k * _B, _B)])

        plsc.subcore_barrier()

        for u in range(4):
            idx_issue(u, u)
        for u in range(2):
            idx_s_wait(u)
            gather_issue(u, u)

        def group(g, carry):
            for j in range(_SLOT):
                b = j % _KR
                bi = j % _KI
                t = g * _SLOT + j
                idx_d_wait(bi)
                gather_wait(bi, b)
                scat_issue(bi, b)
                bp = (b + 2) % _KR
                bip2 = (j + 2) % _KI
                bip4 = (j + 4) % _KI

                def fs(bip2=bip2, bp=bp):
                    scat_wait(bip2, bp)

                if j >= 2:
                    fs()
                else:
                    @pl.when(g > 0)
                    def _fs():
                        fs()
                idx_s_wait(bip2)
                gather_issue(bip2, bp)

                def pi(t=t, bip4=bip4):
                    idx_issue(t + 4, bip4)
                pi()
            return carry
        lax.fori_loop(0, _GROUPS, group, 0)

        for t in range(_GROUPS * _SLOT, _STEPS):
            b = t % _KR
            bi = t % _KI
            idx_d_wait(bi)
            gather_wait(bi, b)
            scat_issue(bi, b)
            scat_wait((bi + 2) % _KI, (b + 2) % _KR)
            if t + 2 < _STEPS:
                idx_s_wait((bi + 2) % _KI)
                gather_issue((bi + 2) % _KI, (b + 2) % _KR)
            if t + 4 < _STEPS:
                idx_issue(t + 4, (bi + 4) % _KI)
        scat_wait(123 % _KI, 123 % _KR)
        scat_wait(124 % _KI, 124 % _KR)

        plsc.subcore_barrier()

        for k in range(_RPT // _B):
            base_r = s * _RPT + k * _B
            pltpu.sync_copy(acc_sh.at[pl.ds(base_r, _B)], rows[0])

            @pl.when(c == 0)
            def _w0():
                pltpu.sync_copy(rows[0], sum0.at[pl.ds(base_r, _B)])

            @pl.when(c != 0)
            def _w1():
                pltpu.sync_copy(rows[0], sum1.at[pl.ds(base_r, _B)])

    return pl.kernel(body, out_type=outs, mesh=_mesh(),
                     scratch_types=scratch)


def _make_deg():
    outs = (jax.ShapeDtypeStruct((_NPAD, _F), jnp.float32),
            jax.ShapeDtypeStruct((_NPAD, _F), jnp.float32))
    scratch = (
        (pltpu.VMEM((_B,), jnp.int32),) * _KI
        + (pltpu.VMEM((_B, _F), jnp.float32),)
        + (pltpu.VMEM_SHARED((_NPAD, _F), jnp.float32),)
        + (pltpu.SemaphoreType.DMA,) * (_KI + _KR)
    )

    def body(dst, deg0, deg1, *rest):
        idd = rest[0:_KI]
        ones_r = rest[_KI]
        acc_sh = rest[_KI + 1]
        sem_id = rest[_KI + 2:_KI + 2 + _KI]
        sem_s = rest[_KI + 2 + _KI:]
        c = lax.axis_index("c")
        s = lax.axis_index("s")
        w = c * _NS + s
        ebase = w * _EPT

        def idx_issue(t, bi):
            pltpu.async_copy(dst.at[pl.ds(ebase + t * _B, _B)], idd[bi],
                             sem_id[bi])

        def idx_wait(bi):
            pltpu.make_async_copy(dst.at[pl.ds(ebase, _B)], idd[bi],
                                  sem_id[bi]).wait()

        def scat_issue(bi, b):
            pltpu.async_copy(ones_r, acc_sh.at[idd[bi]], sem_s[b], add=True)

        def scat_wait(bi, b):
            pltpu.make_async_copy(ones_r, acc_sh.at[idd[bi]],
                                  sem_s[b]).wait()

        _zero_buf(ones_r, _B)
        for k in range(_RPT // _B):
            pltpu.sync_copy(ones_r, acc_sh.at[pl.ds(s * _RPT + k * _B, _B)])
        _zero_buf(ones_r, _B, val=1.0)

        plsc.subcore_barrier()

        for u in range(3):
            idx_issue(u, u)

        def group(g, carry):
            for j in range(_SLOT):
                b = j % _KR
                bi = j % _KI
                t = g * _SLOT + j
                idx_wait(bi)
                scat_issue(bi, b)

                def fs(bi3=(j + 1) % _KI, b3=(b + 1) % _KR):
                    scat_wait(bi3, b3)

                if j >= 3:
                    fs()
                else:
                    @pl.when(g > 0)
                    def _fs():
                        fs()

                def pi(t=t, bi3=(j + 3) % _KI):
                    idx_issue(t + 3, bi3)
                pi()
            return carry
        lax.fori_loop(0, _GROUPS, group, 0)

        for t in range(_GROUPS * _SLOT, _STEPS):
            idx_wait(t % _KI)
            scat_issue(t % _KI, t % _KR)
            scat_wait((t - 3) % _KI, (t - 3) % _KR)
            if t + 3 < _STEPS:
                idx_issue(t + 3, (t + 3) % _KI)
        for u in (122, 123, 124):
            scat_wait(u % _KI, u % _KR)

        plsc.subcore_barrier()

        for k in range(_RPT // _B):
            base_r = s * _RPT + k * _B
            pltpu.sync_copy(acc_sh.at[pl.ds(base_r, _B)], ones_r)

            @pl.when(c == 0)
            def _w0():
                pltpu.sync_copy(ones_r, deg0.at[pl.ds(base_r, _B)])

            @pl.when(c != 0)
            def _w1():
                pltpu.sync_copy(ones_r, deg1.at[pl.ds(base_r, _B)])

    return pl.kernel(body, out_type=outs, mesh=_mesh(),
                     scratch_types=scratch)


def _make_combine(relu):
    rows = 1000

    def body(sum0_ref, sum1_ref, deg0_ref, deg1_ref, x_ref, wl_ref, bl_ref,
             wr_ref, out_ref):
        recip = 1.0 / jnp.maximum(deg0_ref[...] + deg1_ref[...], 1.0)
        mean = (sum0_ref[...] + sum1_ref[...]) * recip
        acc = lax.dot_general(mean, wl_ref[...], (((1,), (1,)), ((), ())),
                              preferred_element_type=jnp.float32)
        acc = acc + lax.dot_general(x_ref[...], wr_ref[...],
                                    (((1,), (1,)), ((), ())),
                                    preferred_element_type=jnp.float32)
        acc = acc + bl_ref[...]
        out_ref[...] = jnp.maximum(acc, 0.0) if relu else acc

    return pl.pallas_call(
        body,
        grid=(_N // rows,),
        in_specs=[
            pl.BlockSpec((rows, _F), lambda i: (i, 0)),
            pl.BlockSpec((rows, _F), lambda i: (i, 0)),
            pl.BlockSpec((rows, _F), lambda i: (i, 0)),
            pl.BlockSpec((rows, _F), lambda i: (i, 0)),
            pl.BlockSpec((rows, _F), lambda i: (i, 0)),
            pl.BlockSpec((_F, _F), lambda i: (0, 0)),
            pl.BlockSpec((1, _F), lambda i: (0, 0)),
            pl.BlockSpec((_F, _F), lambda i: (0, 0)),
        ],
        out_specs=pl.BlockSpec((rows, _F), lambda i: (i, 0)),
        out_shape=jax.ShapeDtypeStruct((_N, _F), jnp.float32),
    )


_agg = _make_agg()
_deg = _make_deg()
_combine_relu = _make_combine(True)
_combine = _make_combine(False)


def kernel(x, edge_index, W1l, b1l, W1r, W2l, b2l, W2r):
    src = edge_index[0]
    dst = edge_index[1]
    deg0, deg1 = _deg(dst)
    sum0, sum1 = _agg(x, src, dst)
    h = _combine_relu(sum0, sum1, deg0, deg1, x,
                      W1l, b1l.reshape(1, _F), W1r)
    t0, t1 = _agg(h, src, dst)
    out = _combine(t0, t1, deg0, deg1, h,
                   W2l, b2l.reshape(1, _F), W2r)
    return out
